# baseline (speedup 1.0000x reference)
import functools

import jax
import jax.numpy as jnp
from jax import lax
from jax.experimental import pallas as pl
from jax.experimental.pallas import tpu as pltpu
from jax.experimental.pallas import tpu_sc as plsc

_T = 2048
_V = 100000
_NW = 32
_BPW = _T // _NW
_L = 16

_V_PACK = 8192
_V_BLK = 1024


def _pack_body(t_ref, to_ref):
    to_ref[...] = jnp.transpose(t_ref[...])


def _compact_table(tok_table):
    return pl.pallas_call(
        _pack_body,
        grid=(pl.cdiv(_V, _V_PACK),),
        in_specs=[pl.BlockSpec((_V_PACK, 1), lambda j: (j, 0))],
        out_specs=pl.BlockSpec((1, _V_PACK), lambda j: (0, j)),
        out_shape=jax.ShapeDtypeStruct((1, _V), jnp.float32),
    )(tok_table)


@functools.partial(
    pl.kernel,
    out_type=jax.ShapeDtypeStruct((_T,), jnp.float32),
    mesh=plsc.VectorSubcoreMesh(core_axis_name="c", subcore_axis_name="s"),
    scratch_types=[
        pltpu.VMEM((_BPW,), jnp.int32),
        pltpu.VMEM((1, _V), jnp.float32),
        pltpu.VMEM((_BPW,), jnp.float32),
    ],
    compiler_params=pltpu.CompilerParams(needs_layout_passes=False),
)
def _sc_gather(idx_hbm, table_hbm, out_hbm, idx_v, table_v, out_v):
    wid = lax.axis_index("s") * 2 + lax.axis_index("c")
    base = wid * _BPW
    pltpu.sync_copy(idx_hbm.at[pl.ds(base, _BPW)], idx_v)
    pltpu.sync_copy(table_hbm, table_v)
    zeros = jnp.zeros((_L,), jnp.int32)
    for k in range(_BPW // _L):
        sl = pl.ds(k * _L, _L)
        vals = plsc.load_gather(table_v, [zeros, idx_v[sl]])
        out_v[sl] = vals
    pltpu.sync_copy(out_v, out_hbm.at[pl.ds(base, _BPW)])


def _proj_body(x_ref, p_ref, w_ref, b_ref, o_ref):
    x_row = x_ref[...].reshape(1, -1)
    x = jnp.transpose(x_row) + p_ref[...]
    w_row = jnp.transpose(w_ref[...])
    b_row = b_ref[...].reshape(1, _V_BLK)
    o_ref[...] = x * w_row + b_row


def kernel(idx, tok_table, pos_emb, W, b):
    T = idx.shape[1]
    V = W.shape[0]
    idx_flat = idx.reshape(T).astype(jnp.int32)

    tok_row = _compact_table(tok_table)
    x_row = _sc_gather(idx_flat, tok_row)

    out = pl.pallas_call(
        _proj_body,
        grid=(pl.cdiv(V, _V_BLK),),
        in_specs=[
            pl.BlockSpec((T,), lambda j: (0,)),
            pl.BlockSpec((T, 1), lambda j: (0, 0)),
            pl.BlockSpec((_V_BLK, 1), lambda j: (j, 0)),
            pl.BlockSpec((_V_BLK,), lambda j: (j,)),
        ],
        out_specs=pl.BlockSpec((T, _V_BLK), lambda j: (0, j)),
        out_shape=jax.ShapeDtypeStruct((T, V), jnp.float32),
        compiler_params=pltpu.CompilerParams(
            dimension_semantics=("arbitrary",),
        ),
    )(x_row, pos_emb, W, b)
    return out.reshape(1, T, V)
